# baseline (speedup 1.0000x reference)
import jax
import jax.numpy as jnp
from jax.experimental import pallas as pl
from jax.experimental.pallas import tpu as pltpu

_NUM_EXPERTS = 16
_TOP_K = 8
_D_MODEL = 2048
_D_FF = 1024
_SHARED_D_FF = 2048
_SCALE = 2.5

_S_BLOCK = 128
_S_STEPS = _SHARED_D_FF // _S_BLOCK
_N_STEPS = _TOP_K + _S_STEPS


def _router_body(x_ref, gw_ref, bias_ref, idx_ref, wts_ref):
    xv = x_ref[...]
    logits = jax.lax.dot_general(
        xv, gw_ref[...], (((1,), (1,)), ((), ())),
        preferred_element_type=jnp.float32)
    scores = jax.nn.sigmoid(logits)
    biased = scores + bias_ref[...]
    iota = jax.lax.broadcasted_iota(jnp.int32, (1, _NUM_EXPERTS), 1)
    neg_inf = jnp.float32(-jnp.inf)
    b = biased
    sel_scores = []
    for r in range(_TOP_K):
        m = jnp.max(b)
        is_m = b == m
        e = jnp.min(jnp.where(is_m, iota, _NUM_EXPERTS))
        onehot = iota == e
        idx_ref[0, r] = e.astype(jnp.int32)
        sel_scores.append(jnp.sum(jnp.where(onehot, scores, 0.0)))
        b = jnp.where(onehot, neg_inf, b)
    total = sel_scores[0]
    for r in range(1, _TOP_K):
        total = total + sel_scores[r]
    inv = _SCALE / (total + 1e-20)
    for r in range(_TOP_K):
        wts_ref[0, r] = sel_scores[r] * inv


def _main_body(idx_ref, wts_ref, x_ref, w1_ref, w3_ref, w2_ref,
               sw1_ref, sw3_ref, sw2_ref, out_ref):
    k = pl.program_id(0)

    @pl.when(k == 0)
    def _init():
        out_ref[...] = jnp.zeros_like(out_ref)

    xv = x_ref[...]

    @pl.when(k < _TOP_K)
    def _routed():
        g = jax.lax.dot_general(
            xv, w1_ref[0], (((1,), (1,)), ((), ())),
            preferred_element_type=jnp.float32)
        u = jax.lax.dot_general(
            xv, w3_ref[0], (((1,), (1,)), ((), ())),
            preferred_element_type=jnp.float32)
        h = (g * jax.nn.sigmoid(g)) * u * wts_ref[0, jnp.minimum(k, _TOP_K - 1)]
        out_ref[...] += jax.lax.dot_general(
            h, w2_ref[0], (((1,), (1,)), ((), ())),
            preferred_element_type=jnp.float32)

    @pl.when(k >= _TOP_K)
    def _shared():
        g = jax.lax.dot_general(
            xv, sw1_ref[...], (((1,), (1,)), ((), ())),
            preferred_element_type=jnp.float32)
        u = jax.lax.dot_general(
            xv, sw3_ref[...], (((1,), (1,)), ((), ())),
            preferred_element_type=jnp.float32)
        h = (g * jax.nn.sigmoid(g)) * u
        out_ref[...] += jax.lax.dot_general(
            h, sw2_ref[...], (((1,), (1,)), ((), ())),
            preferred_element_type=jnp.float32)


def _routed_e(k, idx_ref):
    return idx_ref[0, jnp.minimum(k, _TOP_K - 1)]


def _we_map(k, idx_ref, wts_ref):
    return (_routed_e(k, idx_ref), 0, 0)


def _shared_row(k):
    return jnp.maximum(k - _TOP_K, 0)


def _sw1_map(k, idx_ref, wts_ref):
    return (_shared_row(k), 0)


def _sw2_map(k, idx_ref, wts_ref):
    return (0, _shared_row(k))


@jax.jit
def _run(x, gate_weight, bias, w1, w2, w3, shared_w1, shared_w2, shared_w3):
    xf = x.reshape(1, _D_MODEL)
    bias2 = bias.reshape(1, _NUM_EXPERTS)

    idx, wts = pl.pallas_call(
        _router_body,
        out_shape=(
            jax.ShapeDtypeStruct((1, _TOP_K), jnp.int32),
            jax.ShapeDtypeStruct((1, _TOP_K), jnp.float32),
        ),
        out_specs=(
            pl.BlockSpec(memory_space=pltpu.SMEM),
            pl.BlockSpec(memory_space=pltpu.SMEM),
        ),
    )(xf, gate_weight, bias2)

    out = pl.pallas_call(
        _main_body,
        grid_spec=pltpu.PrefetchScalarGridSpec(
            num_scalar_prefetch=2,
            grid=(_N_STEPS,),
            in_specs=[
                pl.BlockSpec((1, _D_MODEL), lambda k, i, w: (0, 0)),
                pl.BlockSpec((1, _D_FF, _D_MODEL), _we_map),
                pl.BlockSpec((1, _D_FF, _D_MODEL), _we_map),
                pl.BlockSpec((1, _D_MODEL, _D_FF), _we_map),
                pl.BlockSpec((_S_BLOCK, _D_MODEL), _sw1_map),
                pl.BlockSpec((_S_BLOCK, _D_MODEL), _sw1_map),
                pl.BlockSpec((_D_MODEL, _S_BLOCK), _sw2_map),
            ],
            out_specs=pl.BlockSpec((1, _D_MODEL), lambda k, i, w: (0, 0)),
        ),
        out_shape=jax.ShapeDtypeStruct((1, _D_MODEL), jnp.float32),
    )(idx, wts, xf, w1, w3, w2, shared_w1, shared_w3, shared_w2)

    return out.reshape(1, 1, 1, _D_MODEL)


def kernel(x, gate_weight, bias, w1, w2, w3, shared_w1, shared_w2, shared_w3):
    return _run(x, gate_weight, bias, w1, w2, w3,
                shared_w1, shared_w2, shared_w3)


# speedup vs baseline: 1.0812x; 1.0812x over previous
import jax
import jax.numpy as jnp
from jax.experimental import pallas as pl
from jax.experimental.pallas import tpu as pltpu

_NUM_EXPERTS = 16
_TOP_K = 8
_D_MODEL = 2048
_D_FF = 1024
_SHARED_D_FF = 2048
_SCALE = 2.5

_F_BLOCK = 512
_F_BLOCKS = _D_FF // _F_BLOCK
_S_BLOCK = 512
_S_STEPS = _SHARED_D_FF // _S_BLOCK


def _shared_router_body(x_ref, gw_ref, bias_ref, sw1_ref, sw3_ref, sw2_ref,
                        out_ref, idx_ref, wts_ref):
    s = pl.program_id(0)
    xv = x_ref[...]

    @pl.when(s == 0)
    def _router():
        out_ref[...] = jnp.zeros_like(out_ref)
        logits = jax.lax.dot_general(
            xv, gw_ref[...], (((1,), (1,)), ((), ())),
            preferred_element_type=jnp.float32)
        scores = jax.nn.sigmoid(logits)
        biased = scores + bias_ref[...]
        iota = jax.lax.broadcasted_iota(jnp.int32, (1, _NUM_EXPERTS), 1)
        neg_inf = jnp.float32(-jnp.inf)
        b = biased
        sel = []
        for r in range(_TOP_K):
            m = jnp.max(b)
            e = jnp.min(jnp.where(b == m, iota, _NUM_EXPERTS))
            onehot = iota == e
            idx_ref[0, r] = e.astype(jnp.int32)
            sel.append(jnp.sum(jnp.where(onehot, scores, 0.0)))
            b = jnp.where(onehot, neg_inf, b)
        total = sel[0]
        for r in range(1, _TOP_K):
            total = total + sel[r]
        inv = _SCALE / (total + 1e-20)
        for r in range(_TOP_K):
            wts_ref[0, r] = sel[r] * inv

    g = jax.lax.dot_general(
        xv, sw1_ref[...], (((1,), (1,)), ((), ())),
        preferred_element_type=jnp.float32)
    u = jax.lax.dot_general(
        xv, sw3_ref[...], (((1,), (1,)), ((), ())),
        preferred_element_type=jnp.float32)
    h = (g * jax.nn.sigmoid(g)) * u
    out_ref[...] += jax.lax.dot_general(
        h, sw2_ref[...], (((1,), (1,)), ((), ())),
        preferred_element_type=jnp.float32)


def _routed_body(idx_ref, wts_ref, x_ref, shared_ref, w1_ref, w3_ref, w2_ref,
                 out_ref):
    k = pl.program_id(0)
    f = pl.program_id(1)

    @pl.when((k == 0) & (f == 0))
    def _init():
        out_ref[...] = shared_ref[...]

    xv = x_ref[...]
    g = jax.lax.dot_general(
        xv, w1_ref[0], (((1,), (1,)), ((), ())),
        preferred_element_type=jnp.float32)
    u = jax.lax.dot_general(
        xv, w3_ref[0], (((1,), (1,)), ((), ())),
        preferred_element_type=jnp.float32)
    h = (g * jax.nn.sigmoid(g)) * u * wts_ref[0, k]
    out_ref[...] += jax.lax.dot_general(
        h, w2_ref[0], (((1,), (1,)), ((), ())),
        preferred_element_type=jnp.float32)


def _w1_map(k, f, idx_ref, wts_ref):
    return (idx_ref[0, k], f, 0)


def _w2_map(k, f, idx_ref, wts_ref):
    return (idx_ref[0, k], 0, f)


@jax.jit
def _run(x, gate_weight, bias, w1, w2, w3, shared_w1, shared_w2, shared_w3):
    xf = x.reshape(1, _D_MODEL)
    bias2 = bias.reshape(1, _NUM_EXPERTS)

    shared_out, idx, wts = pl.pallas_call(
        _shared_router_body,
        grid=(_S_STEPS,),
        in_specs=[
            pl.BlockSpec((1, _D_MODEL), lambda s: (0, 0)),
            pl.BlockSpec((_NUM_EXPERTS, _D_MODEL), lambda s: (0, 0)),
            pl.BlockSpec((1, _NUM_EXPERTS), lambda s: (0, 0)),
            pl.BlockSpec((_S_BLOCK, _D_MODEL), lambda s: (s, 0)),
            pl.BlockSpec((_S_BLOCK, _D_MODEL), lambda s: (s, 0)),
            pl.BlockSpec((_D_MODEL, _S_BLOCK), lambda s: (0, s)),
        ],
        out_specs=(
            pl.BlockSpec((1, _D_MODEL), lambda s: (0, 0)),
            pl.BlockSpec(memory_space=pltpu.SMEM),
            pl.BlockSpec(memory_space=pltpu.SMEM),
        ),
        out_shape=(
            jax.ShapeDtypeStruct((1, _D_MODEL), jnp.float32),
            jax.ShapeDtypeStruct((1, _TOP_K), jnp.int32),
            jax.ShapeDtypeStruct((1, _TOP_K), jnp.float32),
        ),
    )(xf, gate_weight, bias2, shared_w1, shared_w3, shared_w2)

    out = pl.pallas_call(
        _routed_body,
        grid_spec=pltpu.PrefetchScalarGridSpec(
            num_scalar_prefetch=2,
            grid=(_TOP_K, _F_BLOCKS),
            in_specs=[
                pl.BlockSpec((1, _D_MODEL), lambda k, f, i, w: (0, 0)),
                pl.BlockSpec((1, _D_MODEL), lambda k, f, i, w: (0, 0)),
                pl.BlockSpec((1, _F_BLOCK, _D_MODEL), _w1_map),
                pl.BlockSpec((1, _F_BLOCK, _D_MODEL), _w1_map),
                pl.BlockSpec((1, _D_MODEL, _F_BLOCK), _w2_map),
            ],
            out_specs=pl.BlockSpec((1, _D_MODEL), lambda k, f, i, w: (0, 0)),
        ),
        out_shape=jax.ShapeDtypeStruct((1, _D_MODEL), jnp.float32),
    )(idx, wts, xf, shared_out, w1, w3, w2)

    return out.reshape(1, 1, 1, _D_MODEL)


def kernel(x, gate_weight, bias, w1, w2, w3, shared_w1, shared_w2, shared_w3):
    return _run(x, gate_weight, bias, w1, w2, w3,
                shared_w1, shared_w2, shared_w3)
